# baseline (speedup 1.0000x reference)
import functools

import jax
import jax.numpy as jnp
from jax import lax
from jax.experimental import pallas as pl
from jax.experimental.pallas import tpu as pltpu
from jax.experimental.pallas import tpu_sc as plsc

N = 10000
E = 160000
H = 512
G = 128
L = 5

NPAD = 10240
SCH = 4096
NSC = 40
EPAD = NSC * SCH
OWN = 320
HALF = 160
ACC_R = 168
TRASH = HALF
FB = 2048
LCAP = EPAD + FB
MB = 512

_SC_PARAMS = pltpu.CompilerParams(needs_layout_passes=False)
_SC_MESH = dict(core_axis_name="c", subcore_axis_name="s")


def _bucket_fn():
    mesh = plsc.VectorSubcoreMesh(**_SC_MESH)

    @functools.partial(
        pl.kernel,
        mesh=mesh,
        out_type=(
            jax.ShapeDtypeStruct((64 * LCAP,), jnp.int32),
            jax.ShapeDtypeStruct((512,), jnp.int32),
        ),
        compiler_params=_SC_PARAMS,
        scratch_types=[
            pltpu.VMEM((SCH,), jnp.int32),
            pltpu.VMEM((SCH,), jnp.int32),
            pltpu.VMEM((FB + 16,), jnp.int32),
            pltpu.VMEM((FB + 16,), jnp.int32),
            pltpu.VMEM((16,), jnp.int32),
        ],
    )
    def bucket(src_hbm, dst_hbm, list_hbm, cnt_hbm,
               sbuf, dbuf, ob0, ob1, cb):
        cid = lax.axis_index("c")
        sid = lax.axis_index("s")
        wid = sid * 2 + cid
        one = jnp.ones((16,), jnp.int32)
        zero = jnp.zeros((16,), jnp.int32)
        iota = lax.iota(jnp.int32, 16)
        trash16 = jnp.full((16,), TRASH, jnp.int32)

        def sc_body(si, carry):
            pltpu.sync_copy(src_hbm.at[pl.ds(si * SCH, SCH)], sbuf)
            pltpu.sync_copy(dst_hbm.at[pl.ds(si * SCH, SCH)], dbuf)

            def blk(b, carry):
                s16 = sbuf[pl.ds(b * 16, 16)]
                d16 = dbuf[pl.ds(b * 16, 16)]
                own = lax.shift_right_logical(d16 * 6554, 21)
                mo = own == wid
                nhit = jnp.sum(jnp.where(mo, one, zero))

                def hit(carry):
                    c0, f0, c1, f1 = carry
                    ldst = d16 - own * OWN
                    mh = ldst >= HALF
                    pk = s16 * 512 + ldst - jnp.where(mh, HALF, 0)
                    m0 = mo & jnp.logical_not(mh)
                    m1 = mo & mh

                    mi0 = jnp.where(m0, one, zero)
                    pc0 = plsc.cumsum(mi0)
                    plsc.store_scatter(ob0, [c0 + pc0 - mi0], pk, mask=m0)
                    c0 = c0 + jnp.sum(mi0)
                    mi1 = jnp.where(m1, one, zero)
                    pc1 = plsc.cumsum(mi1)
                    plsc.store_scatter(ob1, [c1 + pc1 - mi1], pk, mask=m1)
                    c1 = c1 + jnp.sum(mi1)

                    fl0 = c0 >= FB

                    @pl.when(fl0)
                    def _():
                        pltpu.sync_copy(
                            ob0.at[pl.ds(0, FB)],
                            list_hbm.at[pl.ds(pl.multiple_of(
                                2 * wid * LCAP + f0, FB), FB)])
                        ob0[pl.ds(0, 16)] = ob0[pl.ds(FB, 16)]

                    c0 = jnp.where(fl0, c0 - FB, c0)
                    f0 = jnp.where(fl0, f0 + FB, f0)

                    fl1 = c1 >= FB

                    @pl.when(fl1)
                    def _():
                        pltpu.sync_copy(
                            ob1.at[pl.ds(0, FB)],
                            list_hbm.at[pl.ds(pl.multiple_of(
                                (2 * wid + 1) * LCAP + f1, FB), FB)])
                        ob1[pl.ds(0, 16)] = ob1[pl.ds(FB, 16)]

                    c1 = jnp.where(fl1, c1 - FB, c1)
                    f1 = jnp.where(fl1, f1 + FB, f1)
                    return (c0, f0, c1, f1)

                return lax.cond(nhit > 0, hit, lambda c: c, carry)

            return lax.fori_loop(0, SCH // 16, blk, carry)

        z = jnp.int32(0)
        c0, f0, c1, f1 = lax.fori_loop(0, NSC, sc_body, (z, z, z, z))

        ob0[pl.ds(c0, 16)] = trash16
        ob1[pl.ds(c1, 16)] = trash16
        pltpu.sync_copy(ob0.at[pl.ds(0, FB)],
                        list_hbm.at[pl.ds(pl.multiple_of(
                            2 * wid * LCAP + f0, FB), FB)])
        pltpu.sync_copy(ob1.at[pl.ds(0, FB)],
                        list_hbm.at[pl.ds(pl.multiple_of(
                            (2 * wid + 1) * LCAP + f1, FB), FB)])
        n0 = f0 + c0
        n1 = f1 + c1
        cb[pl.ds(0, 16)] = jnp.where(
            iota == 0, zero + n0, jnp.where(iota == 1, zero + n1, zero))
        pltpu.sync_copy(cb, cnt_hbm.at[pl.ds(wid * 16, 16)])

    return bucket


_bucket_kernel = None


def _bucket(src, dst):
    global _bucket_kernel
    if _bucket_kernel is None:
        _bucket_kernel = _bucket_fn()
    return _bucket_kernel(src, dst)


@functools.cache
def _seg_fn(d):
    mesh = plsc.VectorSubcoreMesh(**_SC_MESH)

    @functools.partial(
        pl.kernel,
        mesh=mesh,
        out_type=jax.ShapeDtypeStruct((NPAD, d), jnp.float32),
        compiler_params=_SC_PARAMS,
        scratch_types=[
            pltpu.VMEM((FB,), jnp.int32),
            pltpu.VMEM((FB,), jnp.int32),
            pltpu.VMEM((16, d), jnp.float32),
            pltpu.VMEM((16, d), jnp.float32),
            pltpu.VMEM((ACC_R, d), jnp.float32),
            pltpu.VMEM((16,), jnp.int32),
            pltpu.SemaphoreType.DMA,
            pltpu.SemaphoreType.DMA,
            pltpu.SemaphoreType.DMA,
            pltpu.SemaphoreType.DMA,
        ],
    )
    def seg(h_hbm, list_hbm, cnt_hbm, out_hbm,
            lbufa, lbufb, rowsa, rowsb, acc, cbuf,
            seml0, seml1, semg0, semg1):
        lbufs = (lbufa, lbufb)
        rowss = (rowsa, rowsb)
        cid = lax.axis_index("c")
        sid = lax.axis_index("s")
        wid = sid * 2 + cid
        zf = jnp.zeros((16,), jnp.float32)
        zi = jnp.zeros((16,), jnp.int32)
        iota = lax.iota(jnp.int32, 16)
        cols = [q * 16 + iota for q in range(d // 16)]
        semls = (seml0, seml1)
        semgs = (semg0, semg1)

        pltpu.sync_copy(cnt_hbm.at[pl.ds(wid * 16, 16)], cbuf)
        cv = cbuf[pl.ds(0, 16)]

        for hh in range(2):
            row_l = 2 * wid + hh
            n = jnp.sum(jnp.where(iota == hh, cv, zi))

            def zrow(r, _):
                for q in range(d // 16):
                    acc[r, pl.ds(q * 16, 16)] = zf
                return 0

            lax.fori_loop(0, ACC_R, zrow, 0)

            nck = (n + FB - 1) // FB

            def issue_chunk(ck, j, row_l=row_l):
                pltpu.async_copy(
                    list_hbm.at[pl.ds(pl.multiple_of(
                        row_l * LCAP + ck * FB, FB), FB)],
                    lbufs[j], semls[j])

            def wait_chunk(ck, j, row_l=row_l):
                pltpu.make_async_copy(
                    list_hbm.at[pl.ds(pl.multiple_of(
                        row_l * LCAP + ck * FB, FB), FB)],
                    lbufs[j], semls[j]).wait()

            def sv_of(j, g):
                return lax.shift_right_logical(
                    lbufs[j][pl.ds(g * 16, 16)], 9)

            def issue_g(j, jg, g):
                pltpu.async_copy(h_hbm.at[sv_of(j, g)], rowss[jg],
                                 semgs[jg])

            def wait_g(j, jg, g):
                pltpu.make_async_copy(h_hbm.at[sv_of(j, g)], rowss[jg],
                                      semgs[jg]).wait()

            def process(j, jg, g):
                @plsc.parallel_loop(0, 16, unroll=3)
                def _(e):
                    rsp = plsc.load_gather(lbufs[j], [zi + (g * 16 + e)])
                    rsp = lax.rem(rsp, 512)
                    for q in range(d // 16):
                        x = rowss[jg][e, pl.ds(q * 16, 16)]
                        plsc.addupdate_scatter(acc, [rsp, cols[q]], x)

            def do_chunk(ck, j):
                wait_chunk(ck, j)

                @pl.when(ck + 2 < nck)
                def _():
                    issue_chunk(ck + 2, j)

                nv = jnp.minimum(n - ck * FB, FB)
                ngc = (nv + 15) // 16

                @pl.when(ngc > 0)
                def _():
                    issue_g(j, 0, 0)

                @pl.when(ngc > 1)
                def _():
                    issue_g(j, 1, 1)

                def gpair(i, _):
                    g0 = i * 2
                    wait_g(j, 0, g0)
                    process(j, 0, g0)

                    @pl.when(g0 + 2 < ngc)
                    def _():
                        issue_g(j, 0, g0 + 2)

                    @pl.when(g0 + 1 < ngc)
                    def _():
                        wait_g(j, 1, g0 + 1)
                        process(j, 1, g0 + 1)

                        @pl.when(g0 + 3 < ngc)
                        def _():
                            issue_g(j, 1, g0 + 3)

                    return 0

                lax.fori_loop(0, (ngc + 1) // 2, gpair, 0)

            @pl.when(nck > 0)
            def _():
                issue_chunk(0, 0)

            @pl.when(nck > 1)
            def _():
                issue_chunk(1, 1)

            def cpair(i, _):
                do_chunk(i * 2, 0)

                @pl.when(i * 2 + 1 < nck)
                def _():
                    do_chunk(i * 2 + 1, 1)

                return 0

            lax.fori_loop(0, (nck + 1) // 2, cpair, 0)

            pltpu.sync_copy(
                acc.at[pl.ds(0, HALF)],
                out_hbm.at[pl.ds(wid * OWN + hh * HALF, HALF)])

    return seg


def _seg_sum(h, lists, counts):
    return _seg_fn(h.shape[1])(h, lists, counts)


def _mlp(h, msg, w1, b1, w2, b2):
    d = h.shape[1]

    def body(h_ref, m_ref, w1_ref, b1_ref, w2_ref, b2_ref, o_ref):
        a = h_ref[...] + m_ref[...]
        z = jnp.dot(a, w1_ref[...], preferred_element_type=jnp.float32)
        z = jnp.maximum(z + b1_ref[...], 0.0)
        z = jnp.dot(z, w2_ref[...], preferred_element_type=jnp.float32)
        o_ref[...] = jnp.maximum(z + b2_ref[...], 0.0)

    return pl.pallas_call(
        body,
        grid=(NPAD // MB,),
        in_specs=[
            pl.BlockSpec((MB, d), lambda i: (i, 0)),
            pl.BlockSpec((MB, d), lambda i: (i, 0)),
            pl.BlockSpec((d, H), lambda i: (0, 0)),
            pl.BlockSpec((1, H), lambda i: (0, 0)),
            pl.BlockSpec((H, H), lambda i: (0, 0)),
            pl.BlockSpec((1, H), lambda i: (0, 0)),
        ],
        out_specs=pl.BlockSpec((MB, H), lambda i: (i, 0)),
        out_shape=jax.ShapeDtypeStruct((NPAD, H), jnp.float32),
    )(h, msg, w1, b1.reshape(1, H), w2, b2.reshape(1, H))


def _jk_pool(xs, w, b, bt):
    wr = w.reshape(L, H, H)
    btr = bt.reshape(NPAD // MB, 1, MB)

    def body(x0, x1, x2, x3, x4, w_ref, b_ref, bt_ref, o_ref):
        xr = (x0, x1, x2, x3, x4)
        acc = jnp.zeros((MB, H), jnp.float32)
        for l in range(L):
            acc = acc + jnp.dot(xr[l][...], w_ref[l],
                                preferred_element_type=jnp.float32)
        acc = acc + b_ref[...]
        ids = bt_ref[0, 0, :]
        oh = (lax.broadcasted_iota(jnp.int32, (G, MB), 0)
              == ids[None, :]).astype(jnp.float32)
        contrib = jnp.dot(oh, acc, preferred_element_type=jnp.float32)

        @pl.when(pl.program_id(0) == 0)
        def _():
            o_ref[...] = contrib

        @pl.when(pl.program_id(0) != 0)
        def _():
            o_ref[...] += contrib

    return pl.pallas_call(
        body,
        grid=(NPAD // MB,),
        in_specs=[pl.BlockSpec((MB, H), lambda i: (i, 0))] * L
        + [
            pl.BlockSpec((L, H, H), lambda i: (0, 0, 0)),
            pl.BlockSpec((1, H), lambda i: (0, 0)),
            pl.BlockSpec((1, 1, MB), lambda i: (i, 0, 0)),
        ],
        out_specs=pl.BlockSpec((G, H), lambda i: (0, 0)),
        out_shape=jax.ShapeDtypeStruct((G, H), jnp.float32),
        compiler_params=pltpu.CompilerParams(
            dimension_semantics=("arbitrary",)),
    )(*xs, wr, b.reshape(1, H), btr)


def kernel(x, params, edge_index, batch):
    src = jnp.pad(edge_index[0], (0, EPAD - E))
    dst = jnp.pad(edge_index[1], (0, EPAD - E), constant_values=NPAD)
    h = jnp.pad(x, ((0, NPAD - N), (0, 0)))
    bt = jnp.pad(batch, (0, NPAD - N), constant_values=G)

    lists, counts = _bucket(src, dst)
    xs = []
    for l in range(L):
        p = params[f"layer{l}"]
        msg = _seg_sum(h, lists, counts)
        h = _mlp(h, msg, p["W1"], p["b1"], p["W2"], p["b2"])
        xs.append(h)
    return _jk_pool(xs, params["jk"]["W"], params["jk"]["b"], bt)


# speedup vs baseline: 1.0386x; 1.0386x over previous
import functools

import jax
import jax.numpy as jnp
from jax import lax
from jax.experimental import pallas as pl
from jax.experimental.pallas import tpu as pltpu
from jax.experimental.pallas import tpu_sc as plsc

N = 10000
E = 160000
H = 512
G = 128
L = 5

NPAD = 10240
SCH = 4096
NSC = 40
EPAD = NSC * SCH
OWN = 320
HALF = 160
ACC_R = 168
TRASH = HALF
FB = 2048
LCAP = EPAD + FB
MB = 512

_SC_PARAMS = pltpu.CompilerParams(needs_layout_passes=False)
_SC_MESH = dict(core_axis_name="c", subcore_axis_name="s")


def _bucket_fn():
    mesh = plsc.VectorSubcoreMesh(**_SC_MESH)

    @functools.partial(
        pl.kernel,
        mesh=mesh,
        out_type=(
            jax.ShapeDtypeStruct((64 * LCAP,), jnp.int32),
            jax.ShapeDtypeStruct((512,), jnp.int32),
        ),
        compiler_params=_SC_PARAMS,
        scratch_types=[
            pltpu.VMEM((SCH,), jnp.int32),
            pltpu.VMEM((SCH,), jnp.int32),
            pltpu.VMEM((FB + 16,), jnp.int32),
            pltpu.VMEM((FB + 16,), jnp.int32),
            pltpu.VMEM((16,), jnp.int32),
        ],
    )
    def bucket(src_hbm, dst_hbm, list_hbm, cnt_hbm,
               sbuf, dbuf, ob0, ob1, cb):
        cid = lax.axis_index("c")
        sid = lax.axis_index("s")
        wid = sid * 2 + cid
        one = jnp.ones((16,), jnp.int32)
        zero = jnp.zeros((16,), jnp.int32)
        iota = lax.iota(jnp.int32, 16)
        trash16 = jnp.full((16,), TRASH, jnp.int32)

        def sc_body(si, carry):
            pltpu.sync_copy(src_hbm.at[pl.ds(si * SCH, SCH)], sbuf)
            pltpu.sync_copy(dst_hbm.at[pl.ds(si * SCH, SCH)], dbuf)

            def blk(b, carry):
                s16 = sbuf[pl.ds(b * 16, 16)]
                d16 = dbuf[pl.ds(b * 16, 16)]
                own = lax.shift_right_logical(d16 * 6554, 21)
                mo = own == wid
                nhit = jnp.sum(jnp.where(mo, one, zero))

                def hit(carry):
                    c0, f0, c1, f1 = carry
                    ldst = d16 - own * OWN
                    mh = ldst >= HALF
                    pk = s16 * 512 + ldst - jnp.where(mh, HALF, 0)
                    m0 = mo & jnp.logical_not(mh)
                    m1 = mo & mh

                    mi0 = jnp.where(m0, one, zero)
                    pc0 = plsc.cumsum(mi0)
                    plsc.store_scatter(ob0, [c0 + pc0 - mi0], pk, mask=m0)
                    c0 = c0 + jnp.sum(mi0)
                    mi1 = jnp.where(m1, one, zero)
                    pc1 = plsc.cumsum(mi1)
                    plsc.store_scatter(ob1, [c1 + pc1 - mi1], pk, mask=m1)
                    c1 = c1 + jnp.sum(mi1)

                    fl0 = c0 >= FB

                    @pl.when(fl0)
                    def _():
                        pltpu.sync_copy(
                            ob0.at[pl.ds(0, FB)],
                            list_hbm.at[pl.ds(pl.multiple_of(
                                2 * wid * LCAP + f0, FB), FB)])
                        ob0[pl.ds(0, 16)] = ob0[pl.ds(FB, 16)]

                    c0 = jnp.where(fl0, c0 - FB, c0)
                    f0 = jnp.where(fl0, f0 + FB, f0)

                    fl1 = c1 >= FB

                    @pl.when(fl1)
                    def _():
                        pltpu.sync_copy(
                            ob1.at[pl.ds(0, FB)],
                            list_hbm.at[pl.ds(pl.multiple_of(
                                (2 * wid + 1) * LCAP + f1, FB), FB)])
                        ob1[pl.ds(0, 16)] = ob1[pl.ds(FB, 16)]

                    c1 = jnp.where(fl1, c1 - FB, c1)
                    f1 = jnp.where(fl1, f1 + FB, f1)
                    return (c0, f0, c1, f1)

                return lax.cond(nhit > 0, hit, lambda c: c, carry)

            return lax.fori_loop(0, SCH // 16, blk, carry)

        z = jnp.int32(0)
        c0, f0, c1, f1 = lax.fori_loop(0, NSC, sc_body, (z, z, z, z))

        ob0[pl.ds(c0, 16)] = trash16
        ob1[pl.ds(c1, 16)] = trash16
        pltpu.sync_copy(ob0.at[pl.ds(0, FB)],
                        list_hbm.at[pl.ds(pl.multiple_of(
                            2 * wid * LCAP + f0, FB), FB)])
        pltpu.sync_copy(ob1.at[pl.ds(0, FB)],
                        list_hbm.at[pl.ds(pl.multiple_of(
                            (2 * wid + 1) * LCAP + f1, FB), FB)])
        n0 = f0 + c0
        n1 = f1 + c1
        cb[pl.ds(0, 16)] = jnp.where(
            iota == 0, zero + n0, jnp.where(iota == 1, zero + n1, zero))
        pltpu.sync_copy(cb, cnt_hbm.at[pl.ds(wid * 16, 16)])

    return bucket


_bucket_kernel = None


def _bucket(src, dst):
    global _bucket_kernel
    if _bucket_kernel is None:
        _bucket_kernel = _bucket_fn()
    return _bucket_kernel(src, dst)


@functools.cache
def _seg_fn(d):
    mesh = plsc.VectorSubcoreMesh(**_SC_MESH)

    @functools.partial(
        pl.kernel,
        mesh=mesh,
        out_type=jax.ShapeDtypeStruct((NPAD, d), jnp.float32),
        compiler_params=_SC_PARAMS,
        scratch_types=[
            pltpu.VMEM((FB,), jnp.int32),
            pltpu.VMEM((FB,), jnp.int32),
            pltpu.VMEM((16, d), jnp.float32),
            pltpu.VMEM((16, d), jnp.float32),
            pltpu.VMEM((ACC_R, d), jnp.float32),
            pltpu.VMEM((16,), jnp.int32),
            pltpu.SemaphoreType.DMA,
            pltpu.SemaphoreType.DMA,
            pltpu.SemaphoreType.DMA,
            pltpu.SemaphoreType.DMA,
        ],
    )
    def seg(h_hbm, list_hbm, cnt_hbm, out_hbm,
            lbufa, lbufb, rowsa, rowsb, acc, cbuf,
            seml0, seml1, semg0, semg1):
        lbufs = (lbufa, lbufb)
        rowss = (rowsa, rowsb)
        cid = lax.axis_index("c")
        sid = lax.axis_index("s")
        wid = sid * 2 + cid
        zf = jnp.zeros((16,), jnp.float32)
        zi = jnp.zeros((16,), jnp.int32)
        iota = lax.iota(jnp.int32, 16)
        cols = [q * 16 + iota for q in range(d // 16)]
        semls = (seml0, seml1)
        semgs = (semg0, semg1)

        pltpu.sync_copy(cnt_hbm.at[pl.ds(wid * 16, 16)], cbuf)
        cv = cbuf[pl.ds(0, 16)]

        for hh in range(2):
            row_l = 2 * wid + hh
            n = jnp.sum(jnp.where(iota == hh, cv, zi))

            def zrow(r, _):
                for q in range(d // 16):
                    acc[r, pl.ds(q * 16, 16)] = zf
                return 0

            lax.fori_loop(0, ACC_R, zrow, 0)

            nck = (n + FB - 1) // FB

            def issue_chunk(ck, j, row_l=row_l):
                pltpu.async_copy(
                    list_hbm.at[pl.ds(pl.multiple_of(
                        row_l * LCAP + ck * FB, FB), FB)],
                    lbufs[j], semls[j])

            def wait_chunk(ck, j, row_l=row_l):
                pltpu.make_async_copy(
                    list_hbm.at[pl.ds(pl.multiple_of(
                        row_l * LCAP + ck * FB, FB), FB)],
                    lbufs[j], semls[j]).wait()

            def sv_of(j, g):
                return lax.shift_right_logical(
                    lbufs[j][pl.ds(g * 16, 16)], 9)

            def issue_g(j, jg, g):
                pltpu.async_copy(h_hbm.at[sv_of(j, g)], rowss[jg],
                                 semgs[jg])

            def wait_g(j, jg, g):
                pltpu.make_async_copy(h_hbm.at[sv_of(j, g)], rowss[jg],
                                      semgs[jg]).wait()

            def process(j, jg, g):
                @plsc.parallel_loop(0, 16, unroll=2)
                def _(e):
                    rsp = plsc.load_gather(lbufs[j], [zi + (g * 16 + e)])
                    rsp = lax.rem(rsp, 512)
                    for q in range(d // 16):
                        x = rowss[jg][e, pl.ds(q * 16, 16)]
                        plsc.addupdate_scatter(acc, [rsp, cols[q]], x)

            def do_chunk(ck, j):
                wait_chunk(ck, j)

                @pl.when(ck + 2 < nck)
                def _():
                    issue_chunk(ck + 2, j)

                nv = jnp.minimum(n - ck * FB, FB)
                ngc = (nv + 15) // 16

                @pl.when(ngc > 0)
                def _():
                    issue_g(j, 0, 0)

                @pl.when(ngc > 1)
                def _():
                    issue_g(j, 1, 1)

                def gpair(i, _):
                    g0 = i * 2
                    wait_g(j, 0, g0)
                    process(j, 0, g0)

                    @pl.when(g0 + 2 < ngc)
                    def _():
                        issue_g(j, 0, g0 + 2)

                    @pl.when(g0 + 1 < ngc)
                    def _():
                        wait_g(j, 1, g0 + 1)
                        process(j, 1, g0 + 1)

                        @pl.when(g0 + 3 < ngc)
                        def _():
                            issue_g(j, 1, g0 + 3)

                    return 0

                lax.fori_loop(0, (ngc + 1) // 2, gpair, 0)

            @pl.when(nck > 0)
            def _():
                issue_chunk(0, 0)

            @pl.when(nck > 1)
            def _():
                issue_chunk(1, 1)

            def cpair(i, _):
                do_chunk(i * 2, 0)

                @pl.when(i * 2 + 1 < nck)
                def _():
                    do_chunk(i * 2 + 1, 1)

                return 0

            lax.fori_loop(0, (nck + 1) // 2, cpair, 0)

            pltpu.sync_copy(
                acc.at[pl.ds(0, HALF)],
                out_hbm.at[pl.ds(wid * OWN + hh * HALF, HALF)])

    return seg


def _seg_sum(h, lists, counts):
    return _seg_fn(h.shape[1])(h, lists, counts)


def _mlp(h, msg, w1, b1, w2, b2):
    d = h.shape[1]

    def body(h_ref, m_ref, w1_ref, b1_ref, w2_ref, b2_ref, o_ref):
        a = h_ref[...] + m_ref[...]
        z = jnp.dot(a, w1_ref[...], preferred_element_type=jnp.float32)
        z = jnp.maximum(z + b1_ref[...], 0.0)
        z = jnp.dot(z, w2_ref[...], preferred_element_type=jnp.float32)
        o_ref[...] = jnp.maximum(z + b2_ref[...], 0.0)

    return pl.pallas_call(
        body,
        grid=(NPAD // MB,),
        in_specs=[
            pl.BlockSpec((MB, d), lambda i: (i, 0)),
            pl.BlockSpec((MB, d), lambda i: (i, 0)),
            pl.BlockSpec((d, H), lambda i: (0, 0)),
            pl.BlockSpec((1, H), lambda i: (0, 0)),
            pl.BlockSpec((H, H), lambda i: (0, 0)),
            pl.BlockSpec((1, H), lambda i: (0, 0)),
        ],
        out_specs=pl.BlockSpec((MB, H), lambda i: (i, 0)),
        out_shape=jax.ShapeDtypeStruct((NPAD, H), jnp.float32),
    )(h, msg, w1, b1.reshape(1, H), w2, b2.reshape(1, H))


def _jk_pool(xs, w, b, bt):
    wr = w.reshape(L, H, H)
    btr = bt.reshape(NPAD // MB, 1, MB)

    def body(x0, x1, x2, x3, x4, w_ref, b_ref, bt_ref, o_ref):
        xr = (x0, x1, x2, x3, x4)
        acc = jnp.zeros((MB, H), jnp.float32)
        for l in range(L):
            acc = acc + jnp.dot(xr[l][...], w_ref[l],
                                preferred_element_type=jnp.float32)
        acc = acc + b_ref[...]
        ids = bt_ref[0, 0, :]
        oh = (lax.broadcasted_iota(jnp.int32, (G, MB), 0)
              == ids[None, :]).astype(jnp.float32)
        contrib = jnp.dot(oh, acc, preferred_element_type=jnp.float32)

        @pl.when(pl.program_id(0) == 0)
        def _():
            o_ref[...] = contrib

        @pl.when(pl.program_id(0) != 0)
        def _():
            o_ref[...] += contrib

    return pl.pallas_call(
        body,
        grid=(NPAD // MB,),
        in_specs=[pl.BlockSpec((MB, H), lambda i: (i, 0))] * L
        + [
            pl.BlockSpec((L, H, H), lambda i: (0, 0, 0)),
            pl.BlockSpec((1, H), lambda i: (0, 0)),
            pl.BlockSpec((1, 1, MB), lambda i: (i, 0, 0)),
        ],
        out_specs=pl.BlockSpec((G, H), lambda i: (0, 0)),
        out_shape=jax.ShapeDtypeStruct((G, H), jnp.float32),
        compiler_params=pltpu.CompilerParams(
            dimension_semantics=("arbitrary",)),
    )(*xs, wr, b.reshape(1, H), btr)


def kernel(x, params, edge_index, batch):
    src = jnp.pad(edge_index[0], (0, EPAD - E))
    dst = jnp.pad(edge_index[1], (0, EPAD - E), constant_values=NPAD)
    h = jnp.pad(x, ((0, NPAD - N), (0, 0)))
    bt = jnp.pad(batch, (0, NPAD - N), constant_values=G)

    lists, counts = _bucket(src, dst)
    xs = []
    for l in range(L):
        p = params[f"layer{l}"]
        msg = _seg_sum(h, lists, counts)
        h = _mlp(h, msg, p["W1"], p["b1"], p["W2"], p["b2"])
        xs.append(h)
    return _jk_pool(xs, params["jk"]["W"], params["jk"]["b"], bt)
